# Initial kernel scaffold; baseline (speedup 1.0000x reference)
import jax
import jax.numpy as jnp
from jax.experimental import pallas as pl


def kernel(x_train, y_train, x_test, w_train):
    raise NotImplementedError("write your pallas kernel here")


# speedup vs baseline: 1.6796x; 1.6796x over previous
import functools

import jax
import jax.numpy as jnp
from jax import lax
from jax.experimental import pallas as pl
from jax.experimental.pallas import tpu as pltpu
from jax.experimental.pallas import tpu_sc as plsc

BR = 128
BC = 2048
CH_W = 1024
K = 16

F32_INF = float("inf")
I32_BIG = 1 << 30


def _topk_body(x_test_ref, x_train_ref, out_ref, dist_scratch):
    n_train = x_train_ref.shape[0] * pl.num_programs(1)
    n_chunks = n_train // CH_W
    c = pl.program_id(1)

    a = x_test_ref[...]
    b = x_train_ref[...]
    a2 = jnp.sum(a * a, axis=1, keepdims=True)
    b2 = jnp.sum(b * b, axis=1)[None, :]
    mm = lax.dot_general(a, b, (((1,), (1,)), ((), ())),
                         preferred_element_type=jnp.float32)
    d2 = (a2 + b2) - 2.0 * mm
    dist = jnp.sqrt(jnp.maximum(d2, 0.0))

    per_blk = BC // CH_W
    for j in range(per_blk):
        dist_scratch[c * per_blk + j] = dist[:, j * CH_W:(j + 1) * CH_W]

    @pl.when(c == pl.num_programs(1) - 1)
    def _select():
        iota = lax.broadcasted_iota(jnp.int32, (BR, CH_W), 1)

        def chunk_body(c2, carry):
            m, mi, thr, last = carry
            d_c = dist_scratch[c2]
            io = iota + c2 * CH_W
            elig = (d_c > thr) | ((d_c == thr) & (io > last))
            v = jnp.where(elig, d_c, F32_INF)
            cm = jnp.min(v, axis=1, keepdims=True)
            cmi = jnp.min(jnp.where(v == cm, io, I32_BIG),
                          axis=1, keepdims=True)
            upd = cm < m
            return (jnp.where(upd, cm, m), jnp.where(upd, cmi, mi),
                    thr, last)

        thr = jnp.full((BR, 1), -F32_INF, jnp.float32)
        last = jnp.full((BR, 1), -1, jnp.int32)
        cols = []
        for _k in range(K):
            init = (jnp.full((BR, 1), F32_INF, jnp.float32),
                    jnp.full((BR, 1), I32_BIG, jnp.int32), thr, last)
            m, mi, _, _ = lax.fori_loop(0, n_chunks, chunk_body, init)
            thr, last = m, mi
            cols.append(mi)
        out_ref[...] = jnp.concatenate(cols, axis=1)


def _topk_indices(x_train, x_test):
    n_test, d = x_test.shape
    n_train = x_train.shape[0]
    grid = (n_test // BR, n_train // BC)
    return pl.pallas_call(
        _topk_body,
        grid=grid,
        in_specs=[
            pl.BlockSpec((BR, d), lambda r, c: (r, 0)),
            pl.BlockSpec((BC, d), lambda r, c: (c, 0)),
        ],
        out_specs=pl.BlockSpec((BR, K), lambda r, c: (r, 0)),
        out_shape=jax.ShapeDtypeStruct((n_test, K), jnp.int32),
        scratch_shapes=[pltpu.VMEM((n_train // CH_W, BR, CH_W), jnp.float32)],
        compiler_params=pltpu.CompilerParams(
            dimension_semantics=("parallel", "arbitrary")),
    )(x_test, x_train)


def _vote_argmax(y_train, w_train, idx_flat, n_test, n_classes_pad):
    info = plsc.get_sparse_core_info()
    nc, ns, nl = info.num_cores, info.num_subcores, info.num_lanes
    nw = nc * ns
    rows_per_w = n_test // nw
    n_train = y_train.shape[0]
    n_bins = n_classes_pad
    mesh = plsc.VectorSubcoreMesh(core_axis_name="c", subcore_axis_name="s")

    @functools.partial(
        pl.kernel, mesh=mesh,
        out_type=jax.ShapeDtypeStruct((n_test,), jnp.int32),
        compiler_params=pltpu.CompilerParams(needs_layout_passes=False),
        scratch_types=[
            pltpu.VMEM((n_train,), jnp.int32),
            pltpu.VMEM((n_train,), jnp.float32),
            pltpu.VMEM((rows_per_w * K,), jnp.int32),
            pltpu.VMEM((n_bins,), jnp.float32),
            pltpu.VMEM((rows_per_w,), jnp.int32),
        ],
    )
    def sc_kernel(y_hbm, w_hbm, idx_hbm, out_hbm,
                  y_v, w_v, idx_v, vote_v, res_v):
        wid = lax.axis_index("s") * nc + lax.axis_index("c")
        base = wid * rows_per_w
        pltpu.sync_copy(y_hbm, y_v)
        pltpu.sync_copy(w_hbm, w_v)
        pltpu.sync_copy(idx_hbm.at[pl.ds(base * K, rows_per_w * K)], idx_v)

        zeros16 = jnp.zeros((nl,), jnp.float32)
        lane = lax.broadcasted_iota(jnp.int32, (nl,), 0)

        def zero_body(i, carry):
            vote_v[pl.ds(i * nl, nl)] = zeros16
            return carry

        lax.fori_loop(0, n_bins // nl, zero_body, 0)

        def row_body(r, carry):
            idx_row = idx_v[pl.ds(r * K, K)]
            labels = plsc.load_gather(y_v, [idx_row])
            wts = plsc.load_gather(w_v, [idx_row])
            plsc.addupdate_scatter(vote_v, [labels], wts)

            def amax_body(c, mcarry):
                m, mi = mcarry
                v = vote_v[pl.ds(c * nl, nl)]
                io = lane + c * nl
                upd = v > m
                return (jnp.where(upd, v, m), jnp.where(upd, io, mi))

            m, mi = lax.fori_loop(
                0, n_bins // nl, amax_body,
                (jnp.full((nl,), -F32_INF, jnp.float32),
                 jnp.full((nl,), I32_BIG, jnp.int32)))
            best = jnp.min(jnp.where(m == jnp.max(m), mi, I32_BIG))
            plsc.store_scatter(res_v, [jnp.full((nl,), r, jnp.int32)],
                               jnp.full((nl,), best, jnp.int32),
                               mask=lane == 0)
            plsc.store_scatter(vote_v, [labels], zeros16)
            return carry

        lax.fori_loop(0, rows_per_w, row_body, 0)
        pltpu.sync_copy(res_v, out_hbm.at[pl.ds(base, rows_per_w)])

    return sc_kernel(y_train, w_train, idx_flat)


def kernel(x_train, y_train, x_test, w_train):
    idx = _topk_indices(x_train, x_test)
    return _vote_argmax(y_train, w_train, idx.reshape(-1),
                        x_test.shape[0], 1024)


# speedup vs baseline: 4.9993x; 2.9765x over previous
import functools

import jax
import jax.numpy as jnp
from jax import lax
from jax.experimental import pallas as pl
from jax.experimental.pallas import tpu as pltpu
from jax.experimental.pallas import tpu_sc as plsc

BR = 128
BC = 2048
G = 16
K = 16

F32_INF = float("inf")
I32_BIG = 1 << 30


def _group_body(x_test_ref, x_train_ref, dist_ref, grp_ref, gm_scratch):
    c = pl.program_id(1)
    n_groups = gm_scratch.shape[1]
    gpb = BC // G

    a = x_test_ref[...]
    b = x_train_ref[...]
    a2 = jnp.sum(a * a, axis=1, keepdims=True)
    b2 = jnp.sum(b * b, axis=1)[None, :]
    mm = lax.dot_general(a, b, (((1,), (1,)), ((), ())),
                         preferred_element_type=jnp.float32)
    d2 = (a2 + b2) - 2.0 * mm
    dist = jnp.sqrt(jnp.maximum(d2, 0.0))
    dist_ref[...] = dist

    m = dist
    for s in (1, 2, 4, 8):
        m = jnp.minimum(m, pltpu.roll(m, BC - s, 1))
    io_l = lax.broadcasted_iota(jnp.int32, (BC, gpb), 0)
    io_g = lax.broadcasted_iota(jnp.int32, (BC, gpb), 1)
    sel = (io_l == G * io_g).astype(jnp.float32)
    gm = lax.dot_general(m, sel, (((1,), (0,)), ((), ())),
                         precision=lax.Precision.HIGHEST,
                         preferred_element_type=jnp.float32)
    gm_scratch[:, pl.ds(c * gpb, gpb)] = gm

    @pl.when(c == pl.num_programs(1) - 1)
    def _select():
        g = gm_scratch[...]
        iota_g = lax.broadcasted_iota(jnp.int32, (BR, n_groups), 1)
        thr = jnp.full((BR, 1), -F32_INF, jnp.float32)
        last = jnp.full((BR, 1), -1, jnp.int32)
        cols = []
        for _k in range(K):
            elig = (g > thr) | ((g == thr) & (iota_g > last))
            v = jnp.where(elig, g, F32_INF)
            cm = jnp.min(v, axis=1, keepdims=True)
            cmi = jnp.min(jnp.where(v == cm, iota_g, I32_BIG),
                          axis=1, keepdims=True)
            thr, last = cm, cmi
            cols.append(cmi)
        grp_ref[...] = jnp.concatenate(cols, axis=1)


def _topk_groups(x_train, x_test):
    n_test, d = x_test.shape
    n_train = x_train.shape[0]
    grid = (n_test // BR, n_train // BC)
    return pl.pallas_call(
        _group_body,
        grid=grid,
        in_specs=[
            pl.BlockSpec((BR, d), lambda r, c: (r, 0)),
            pl.BlockSpec((BC, d), lambda r, c: (c, 0)),
        ],
        out_specs=[
            pl.BlockSpec((BR, BC), lambda r, c: (r, c)),
            pl.BlockSpec((BR, K), lambda r, c: (r, 0)),
        ],
        out_shape=[
            jax.ShapeDtypeStruct((n_test, n_train), jnp.float32),
            jax.ShapeDtypeStruct((n_test, K), jnp.int32),
        ],
        scratch_shapes=[pltpu.VMEM((BR, n_train // G), jnp.float32)],
        compiler_params=pltpu.CompilerParams(
            dimension_semantics=("parallel", "arbitrary")),
    )(x_test, x_train)


def _vote_argmax(y_train, w_train, grp_flat, dist, n_test, n_bins):
    info = plsc.get_sparse_core_info()
    nc, ns, nl = info.num_cores, info.num_subcores, info.num_lanes
    nw = nc * ns
    rows_per_w = n_test // nw
    n_train = y_train.shape[0]
    mesh = plsc.VectorSubcoreMesh(core_axis_name="c", subcore_axis_name="s")

    @functools.partial(
        pl.kernel, mesh=mesh,
        out_type=jax.ShapeDtypeStruct((n_test,), jnp.int32),
        compiler_params=pltpu.CompilerParams(needs_layout_passes=False),
        scratch_types=[
            pltpu.VMEM((n_train,), jnp.int32),
            pltpu.VMEM((n_train,), jnp.float32),
            pltpu.VMEM((rows_per_w * K,), jnp.int32),
            pltpu.VMEM((n_train,), jnp.float32),
            pltpu.VMEM((n_bins,), jnp.float32),
            pltpu.VMEM((rows_per_w,), jnp.int32),
        ],
    )
    def sc_kernel(y_hbm, w_hbm, grp_hbm, dist_hbm, out_hbm,
                  y_v, w_v, grp_v, row_v, vote_v, res_v):
        wid = lax.axis_index("s") * nc + lax.axis_index("c")
        base = wid * rows_per_w
        pltpu.sync_copy(y_hbm, y_v)
        pltpu.sync_copy(w_hbm, w_v)
        pltpu.sync_copy(grp_hbm.at[pl.ds(base * K, rows_per_w * K)], grp_v)

        zeros16 = jnp.zeros((nl,), jnp.float32)
        lane = lax.broadcasted_iota(jnp.int32, (nl,), 0)

        def row_body(r, carry):
            pltpu.sync_copy(dist_hbm.at[base + r], row_v)

            def zero_body(i, zc):
                vote_v[pl.ds(i * nl, nl)] = zeros16
                return zc

            lax.fori_loop(0, n_bins // nl, zero_body, 0)

            tvecs, vals = [], []
            for j in range(K):
                gb = plsc.load_gather(
                    grp_v, [jnp.full((nl,), r * K + j, jnp.int32)])
                tvec = gb * G + lane
                tvecs.append(tvec)
                vals.append(plsc.load_gather(row_v, [tvec]))

            acc, _ = plsc.sort_key_val(vals[0], vals[0])
            cnt = jnp.zeros((nl,), jnp.int32)
            for j in range(1, K):
                cj, _ = plsc.sort_key_val(vals[j], vals[j])
                mrg = jnp.minimum(acc, lax.rev(cj, (0,)))
                acc, _ = plsc.sort_key_val(mrg, mrg)
            t = jnp.max(acc)

            for j in range(K):
                cnt = cnt + plsc.all_reduce_population_count(vals[j] <= t)
            n_le = jnp.max(cnt)

            @pl.when(n_le == K)
            def _common():
                for j in range(K):
                    lab = plsc.load_gather(y_v, [tvecs[j]])
                    wt = plsc.load_gather(w_v, [tvecs[j]])
                    plsc.addupdate_scatter(vote_v, [lab], wt,
                                           mask=vals[j] <= t)

            @pl.when(n_le != K)
            def _rare():
                def ext_body(_k, ecarry):
                    thr, lastg = ecarry
                    bv = jnp.full((nl,), F32_INF, jnp.float32)
                    bi = jnp.full((nl,), I32_BIG, jnp.int32)
                    for j in range(K):
                        v = vals[j]
                        tvec = tvecs[j]
                        elig = (v > thr) | ((v == thr) & (tvec > lastg))
                        vv = jnp.where(elig, v, F32_INF)
                        mv = jnp.min(vv)
                        mi = jnp.min(jnp.where(vv == mv, tvec, I32_BIG))
                        take = (mv < bv) | ((mv == bv) & (mi < bi))
                        bv = jnp.where(take, mv, bv)
                        bi = jnp.where(take, mi, bi)
                    lab = plsc.load_gather(y_v, [bi])
                    wt = plsc.load_gather(w_v, [bi])
                    plsc.addupdate_scatter(vote_v, [lab], wt,
                                           mask=lane == 0)
                    return (bv, bi)

                lax.fori_loop(0, K, ext_body,
                              (jnp.full((nl,), -F32_INF, jnp.float32),
                               jnp.full((nl,), -1, jnp.int32)))

            def amax_body(cb, mcarry):
                m, mi = mcarry
                v = vote_v[pl.ds(cb * nl, nl)]
                io = lane + cb * nl
                upd = v > m
                return (jnp.where(upd, v, m), jnp.where(upd, io, mi))

            m, mi = lax.fori_loop(
                0, n_bins // nl, amax_body,
                (jnp.full((nl,), -F32_INF, jnp.float32),
                 jnp.full((nl,), I32_BIG, jnp.int32)))
            best = jnp.min(jnp.where(m == jnp.max(m), mi, I32_BIG))
            plsc.store_scatter(res_v, [jnp.full((nl,), r, jnp.int32)],
                               jnp.full((nl,), best, jnp.int32),
                               mask=lane == 0)
            return carry

        lax.fori_loop(0, rows_per_w, row_body, 0)
        pltpu.sync_copy(res_v, out_hbm.at[pl.ds(base, rows_per_w)])

    return sc_kernel(y_train, w_train, grp_flat, dist)


def kernel(x_train, y_train, x_test, w_train):
    dist, grp = _topk_groups(x_train, x_test)
    return _vote_argmax(y_train, w_train, grp.reshape(-1), dist,
                        x_test.shape[0], 1024)


# speedup vs baseline: 5.9704x; 1.1942x over previous
import functools

import jax
import jax.numpy as jnp
from jax import lax
from jax.experimental import pallas as pl
from jax.experimental.pallas import tpu as pltpu
from jax.experimental.pallas import tpu_sc as plsc

BR = 256
BC = 2048
G = 16
K = 16

F32_INF = float("inf")
I32_BIG = 1 << 30


def _group_body(x_test_ref, x_train_ref, dist_ref, grp_ref,
                gm_scratch, a2_scratch, b2_scratch):
    r = pl.program_id(0)
    c = pl.program_id(1)
    n_groups = gm_scratch.shape[1]
    gpb = BC // G

    a = x_test_ref[...]
    b = x_train_ref[...]

    @pl.when(c == 0)
    def _a_norms():
        a2_scratch[...] = jnp.sum(a * a, axis=1, keepdims=True)

    @pl.when(r == 0)
    def _b_norms():
        b2_scratch[:, pl.ds(c * BC, BC)] = jnp.sum(b * b, axis=1)[None, :]

    a2 = a2_scratch[...]
    b2 = b2_scratch[:, pl.ds(c * BC, BC)]
    mm = lax.dot_general(a, b, (((1,), (1,)), ((), ())),
                         preferred_element_type=jnp.float32)
    d2 = (a2 + b2) - 2.0 * mm
    dist = jnp.sqrt(jnp.maximum(d2, 0.0))
    dist_ref[...] = dist

    m = dist
    for s in (1, 2, 4, 8):
        m = jnp.minimum(m, pltpu.roll(m, BC - s, 1))
    io_l = lax.broadcasted_iota(jnp.int32, (BC, gpb), 0)
    io_g = lax.broadcasted_iota(jnp.int32, (BC, gpb), 1)
    sel = (io_l == G * io_g).astype(jnp.float32)
    gm = lax.dot_general(m, sel, (((1,), (0,)), ((), ())),
                         precision=lax.Precision.HIGHEST,
                         preferred_element_type=jnp.float32)
    gm_scratch[:, pl.ds(c * gpb, gpb)] = gm

    @pl.when(c == pl.num_programs(1) - 1)
    def _select():
        g = gm_scratch[...]
        iota_g = lax.broadcasted_iota(jnp.int32, (BR, n_groups), 1)
        thr = jnp.full((BR, 1), -F32_INF, jnp.float32)
        last = jnp.full((BR, 1), -1, jnp.int32)
        cols = []
        for _k in range(K):
            elig = (g > thr) | ((g == thr) & (iota_g > last))
            v = jnp.where(elig, g, F32_INF)
            cm = jnp.min(v, axis=1, keepdims=True)
            cmi = jnp.min(jnp.where(v == cm, iota_g, I32_BIG),
                          axis=1, keepdims=True)
            thr, last = cm, cmi
            cols.append(cmi)
        grp_ref[...] = jnp.concatenate(cols, axis=1)


def _topk_groups(x_train, x_test):
    n_test, d = x_test.shape
    n_train = x_train.shape[0]
    grid = (n_test // BR, n_train // BC)
    return pl.pallas_call(
        _group_body,
        grid=grid,
        in_specs=[
            pl.BlockSpec((BR, d), lambda r, c: (r, 0)),
            pl.BlockSpec((BC, d), lambda r, c: (c, 0)),
        ],
        out_specs=[
            pl.BlockSpec((BR, BC), lambda r, c: (r, c)),
            pl.BlockSpec((BR, K), lambda r, c: (r, 0)),
        ],
        out_shape=[
            jax.ShapeDtypeStruct((n_test, n_train), jnp.float32),
            jax.ShapeDtypeStruct((n_test, K), jnp.int32),
        ],
        scratch_shapes=[
            pltpu.VMEM((BR, n_train // G), jnp.float32),
            pltpu.VMEM((BR, 1), jnp.float32),
            pltpu.VMEM((1, n_train), jnp.float32),
        ],
        compiler_params=pltpu.CompilerParams(
            dimension_semantics=("arbitrary", "arbitrary")),
    )(x_test, x_train)


def _vote_argmax(y_train, w_train, grp_flat, dist, n_test, n_bins):
    info = plsc.get_sparse_core_info()
    nc, ns, nl = info.num_cores, info.num_subcores, info.num_lanes
    nw = nc * ns
    rows_per_w = n_test // nw
    n_train = y_train.shape[0]
    mesh = plsc.VectorSubcoreMesh(core_axis_name="c", subcore_axis_name="s")

    @functools.partial(
        pl.kernel, mesh=mesh,
        out_type=jax.ShapeDtypeStruct((n_test,), jnp.int32),
        compiler_params=pltpu.CompilerParams(needs_layout_passes=False),
        scratch_types=[
            pltpu.VMEM((n_train,), jnp.int32),
            pltpu.VMEM((n_train,), jnp.float32),
            pltpu.VMEM((rows_per_w * K,), jnp.int32),
            pltpu.VMEM((4, n_train), jnp.float32),
            pltpu.VMEM((n_bins,), jnp.float32),
            pltpu.VMEM((rows_per_w,), jnp.int32),
        ],
    )
    def sc_kernel(y_hbm, w_hbm, grp_hbm, dist_hbm, out_hbm,
                  y_v, w_v, grp_v, row_v, vote_v, res_v):
        wid = lax.axis_index("s") * nc + lax.axis_index("c")
        base = wid * rows_per_w
        pltpu.sync_copy(y_hbm, y_v)
        pltpu.sync_copy(w_hbm, w_v)
        pltpu.sync_copy(grp_hbm.at[pl.ds(base * K, rows_per_w * K)], grp_v)

        zeros16 = jnp.zeros((nl,), jnp.float32)
        lane = lax.broadcasted_iota(jnp.int32, (nl,), 0)

        def batch_body(bt, bcarry):
            pltpu.sync_copy(dist_hbm.at[pl.ds(base + bt * 4, 4)], row_v)
            lax.fori_loop(0, 4, lambda q, c2: row_body(bt * 4 + q, q, c2), 0)
            return bcarry

        def row_body(r, q, carry):
            def zero_body(i, zc):
                vote_v[pl.ds(i * nl, nl)] = zeros16
                return zc

            lax.fori_loop(0, n_bins // nl, zero_body, 0)

            tvecs, vals = [], []
            for j in range(K):
                gb = plsc.load_gather(
                    grp_v, [jnp.full((nl,), r * K + j, jnp.int32)])
                tvec = gb * G + lane
                tvecs.append(tvec)
                vals.append(plsc.load_gather(
                    row_v, [jnp.full((nl,), q, jnp.int32), tvec]))

            acc, _ = plsc.sort_key_val(vals[0], vals[0])
            cnt = jnp.zeros((nl,), jnp.int32)
            for j in range(1, K):
                cj, _ = plsc.sort_key_val(vals[j], vals[j])
                mrg = jnp.minimum(acc, lax.rev(cj, (0,)))
                acc, _ = plsc.sort_key_val(mrg, mrg)
            t = jnp.max(acc)

            for j in range(K):
                cnt = cnt + plsc.all_reduce_population_count(vals[j] <= t)
            n_le = jnp.max(cnt)

            @pl.when(n_le == K)
            def _common():
                for j in range(K):
                    lab = plsc.load_gather(y_v, [tvecs[j]])
                    wt = plsc.load_gather(w_v, [tvecs[j]])
                    plsc.addupdate_scatter(vote_v, [lab], wt,
                                           mask=vals[j] <= t)

            @pl.when(n_le != K)
            def _rare():
                def ext_body(_k, ecarry):
                    thr, lastg = ecarry
                    bv = jnp.full((nl,), F32_INF, jnp.float32)
                    bi = jnp.full((nl,), I32_BIG, jnp.int32)
                    for j in range(K):
                        v = vals[j]
                        tvec = tvecs[j]
                        elig = (v > thr) | ((v == thr) & (tvec > lastg))
                        vv = jnp.where(elig, v, F32_INF)
                        mv = jnp.min(vv)
                        mi = jnp.min(jnp.where(vv == mv, tvec, I32_BIG))
                        take = (mv < bv) | ((mv == bv) & (mi < bi))
                        bv = jnp.where(take, mv, bv)
                        bi = jnp.where(take, mi, bi)
                    lab = plsc.load_gather(y_v, [bi])
                    wt = plsc.load_gather(w_v, [bi])
                    plsc.addupdate_scatter(vote_v, [lab], wt,
                                           mask=lane == 0)
                    return (bv, bi)

                lax.fori_loop(0, K, ext_body,
                              (jnp.full((nl,), -F32_INF, jnp.float32),
                               jnp.full((nl,), -1, jnp.int32)))

            def amax_body(cb, mcarry):
                m, mi = mcarry
                v = vote_v[pl.ds(cb * nl, nl)]
                io = lane + cb * nl
                upd = v > m
                return (jnp.where(upd, v, m), jnp.where(upd, io, mi))

            m, mi = lax.fori_loop(
                0, n_bins // nl, amax_body,
                (jnp.full((nl,), -F32_INF, jnp.float32),
                 jnp.full((nl,), I32_BIG, jnp.int32)))
            best = jnp.min(jnp.where(m == jnp.max(m), mi, I32_BIG))
            plsc.store_scatter(res_v, [jnp.full((nl,), r, jnp.int32)],
                               jnp.full((nl,), best, jnp.int32),
                               mask=lane == 0)
            return carry

        lax.fori_loop(0, rows_per_w // 4, batch_body, 0)
        pltpu.sync_copy(res_v, out_hbm.at[pl.ds(base, rows_per_w)])

    return sc_kernel(y_train, w_train, grp_flat, dist)


def kernel(x_train, y_train, x_test, w_train):
    dist, grp = _topk_groups(x_train, x_test)
    return _vote_argmax(y_train, w_train, grp.reshape(-1), dist,
                        x_test.shape[0], 1024)


# speedup vs baseline: 6.1945x; 1.0375x over previous
import functools

import jax
import jax.numpy as jnp
from jax import lax
from jax.experimental import pallas as pl
from jax.experimental.pallas import tpu as pltpu
from jax.experimental.pallas import tpu_sc as plsc

BR = 256
BC = 2048
G = 16
K = 16

F32_INF = float("inf")
I32_BIG = 1 << 30


def _group_body(x_test_ref, x_train_ref, dist_ref, grp_ref,
                gm_scratch, a2_scratch, b2_scratch):
    r = pl.program_id(0)
    c = pl.program_id(1)
    n_groups = gm_scratch.shape[1]
    gpb = BC // G

    a = x_test_ref[...]
    b = x_train_ref[...]

    @pl.when(c == 0)
    def _a_norms():
        a2_scratch[...] = jnp.sum(a * a, axis=1, keepdims=True)

    @pl.when(r == 0)
    def _b_norms():
        b2_scratch[:, pl.ds(c * BC, BC)] = jnp.sum(b * b, axis=1)[None, :]

    a2 = a2_scratch[...]
    b2 = b2_scratch[:, pl.ds(c * BC, BC)]
    mm = lax.dot_general(a, b, (((1,), (1,)), ((), ())),
                         preferred_element_type=jnp.float32)
    d2 = (a2 + b2) - 2.0 * mm
    dist = jnp.sqrt(jnp.maximum(d2, 0.0))
    dist_ref[...] = dist

    m = dist
    for s in (1, 2, 4, 8):
        m = jnp.minimum(m, pltpu.roll(m, BC - s, 1))
    io_l = lax.broadcasted_iota(jnp.int32, (BC, gpb), 0)
    io_g = lax.broadcasted_iota(jnp.int32, (BC, gpb), 1)
    sel = (io_l == G * io_g).astype(jnp.float32)
    gm = lax.dot_general(m, sel, (((1,), (0,)), ((), ())),
                         precision=lax.Precision.HIGHEST,
                         preferred_element_type=jnp.float32)
    gm_scratch[:, pl.ds(c * gpb, gpb)] = gm

    @pl.when(c == pl.num_programs(1) - 1)
    def _select():
        g = gm_scratch[...]
        iota_g = lax.broadcasted_iota(jnp.int32, (BR, n_groups), 1)
        thr = jnp.full((BR, 1), -F32_INF, jnp.float32)
        last = jnp.full((BR, 1), -1, jnp.int32)
        cols = []
        for _k in range(K):
            elig = (g > thr) | ((g == thr) & (iota_g > last))
            v = jnp.where(elig, g, F32_INF)
            cm = jnp.min(v, axis=1, keepdims=True)
            cmi = jnp.min(jnp.where(v == cm, iota_g, I32_BIG),
                          axis=1, keepdims=True)
            thr, last = cm, cmi
            cols.append(cmi)
        grp_ref[...] = jnp.concatenate(cols, axis=1)


def _topk_groups(x_train, x_test):
    n_test, d = x_test.shape
    n_train = x_train.shape[0]
    grid = (n_test // BR, n_train // BC)
    return pl.pallas_call(
        _group_body,
        grid=grid,
        in_specs=[
            pl.BlockSpec((BR, d), lambda r, c: (r, 0)),
            pl.BlockSpec((BC, d), lambda r, c: (c, 0)),
        ],
        out_specs=[
            pl.BlockSpec((BR, BC), lambda r, c: (r, c)),
            pl.BlockSpec((BR, K), lambda r, c: (r, 0)),
        ],
        out_shape=[
            jax.ShapeDtypeStruct((n_test, n_train), jnp.float32),
            jax.ShapeDtypeStruct((n_test, K), jnp.int32),
        ],
        scratch_shapes=[
            pltpu.VMEM((BR, n_train // G), jnp.float32),
            pltpu.VMEM((BR, 1), jnp.float32),
            pltpu.VMEM((1, n_train), jnp.float32),
        ],
        compiler_params=pltpu.CompilerParams(
            dimension_semantics=("arbitrary", "arbitrary")),
    )(x_test, x_train)


def _vote_argmax(y_train, w_train, grp_flat, dist, n_test, n_bins):
    info = plsc.get_sparse_core_info()
    nc, ns, nl = info.num_cores, info.num_subcores, info.num_lanes
    nw = nc * ns
    rows_per_w = n_test // nw
    n_train = y_train.shape[0]
    mesh = plsc.VectorSubcoreMesh(core_axis_name="c", subcore_axis_name="s")

    @functools.partial(
        pl.kernel, mesh=mesh,
        out_type=jax.ShapeDtypeStruct((n_test,), jnp.int32),
        compiler_params=pltpu.CompilerParams(needs_layout_passes=False),
        scratch_types=[
            pltpu.VMEM((n_train,), jnp.int32),
            pltpu.VMEM((n_train,), jnp.float32),
            pltpu.VMEM((rows_per_w * K,), jnp.int32),
            pltpu.VMEM((4, n_train), jnp.float32),
            pltpu.VMEM((n_bins,), jnp.float32),
            pltpu.VMEM((rows_per_w,), jnp.int32),
        ],
    )
    def sc_kernel(y_hbm, w_hbm, grp_hbm, dist_hbm, out_hbm,
                  y_v, w_v, grp_v, row_v, vote_v, res_v):
        wid = lax.axis_index("s") * nc + lax.axis_index("c")
        base = wid * rows_per_w
        pltpu.sync_copy(y_hbm, y_v)
        pltpu.sync_copy(w_hbm, w_v)
        pltpu.sync_copy(grp_hbm.at[pl.ds(base * K, rows_per_w * K)], grp_v)

        zeros16 = jnp.zeros((nl,), jnp.float32)
        lane = lax.broadcasted_iota(jnp.int32, (nl,), 0)

        def batch_body(bt, bcarry):
            pltpu.sync_copy(dist_hbm.at[pl.ds(base + bt * 4, 4)], row_v)
            lax.fori_loop(0, 4, lambda q, c2: row_body(bt * 4 + q, q, c2), 0)
            return bcarry

        def row_body(r, q, carry):
            def zero_body(i, zc):
                vote_v[pl.ds(i * nl, nl)] = zeros16
                return zc

            lax.fori_loop(0, n_bins // nl, zero_body, 0)

            tvecs, vals = [], []
            for j in range(K):
                gb = plsc.load_gather(
                    grp_v, [jnp.full((nl,), r * K + j, jnp.int32)])
                tvec = gb * G + lane
                tvecs.append(tvec)
                vals.append(plsc.load_gather(
                    row_v, [jnp.full((nl,), q, jnp.int32), tvec]))

            acc, _ = plsc.sort_key_val(vals[0], vals[0])
            cnt = jnp.zeros((nl,), jnp.int32)
            for j in range(1, K):
                cj, _ = plsc.sort_key_val(vals[j], vals[j])
                mrg = jnp.minimum(acc, lax.rev(cj, (0,)))
                acc, _ = plsc.sort_key_val(mrg, mrg)
            t = jnp.max(acc)

            for j in range(K):
                cnt = cnt + plsc.all_reduce_population_count(vals[j] <= t)
            n_le = jnp.max(cnt)

            @pl.when(n_le == K)
            def _common():
                for j in range(K):
                    lab = plsc.load_gather(y_v, [tvecs[j]])
                    wt = plsc.load_gather(w_v, [tvecs[j]])
                    plsc.addupdate_scatter(vote_v, [lab], wt,
                                           mask=vals[j] <= t)

            @pl.when(n_le != K)
            def _rare():
                def ext_body(_k, ecarry):
                    thr, lastg = ecarry
                    bv = jnp.full((nl,), F32_INF, jnp.float32)
                    bi = jnp.full((nl,), I32_BIG, jnp.int32)
                    for j in range(K):
                        v = vals[j]
                        tvec = tvecs[j]
                        elig = (v > thr) | ((v == thr) & (tvec > lastg))
                        vv = jnp.where(elig, v, F32_INF)
                        mv = jnp.min(vv)
                        mi = jnp.min(jnp.where(vv == mv, tvec, I32_BIG))
                        take = (mv < bv) | ((mv == bv) & (mi < bi))
                        bv = jnp.where(take, mv, bv)
                        bi = jnp.where(take, mi, bi)
                    lab = plsc.load_gather(y_v, [bi])
                    wt = plsc.load_gather(w_v, [bi])
                    plsc.addupdate_scatter(vote_v, [lab], wt,
                                           mask=lane == 0)
                    return (bv, bi)

                lax.fori_loop(0, K, ext_body,
                              (jnp.full((nl,), -F32_INF, jnp.float32),
                               jnp.full((nl,), -1, jnp.int32)))

            def amax_body(cb, mcarry):
                m, mi = mcarry
                v = vote_v[pl.ds(cb * nl, nl)]
                io = lane + cb * nl
                upd = v > m
                return (jnp.where(upd, v, m), jnp.where(upd, io, mi))

            m, mi = lax.fori_loop(
                0, n_bins // nl, amax_body,
                (jnp.full((nl,), -F32_INF, jnp.float32),
                 jnp.full((nl,), I32_BIG, jnp.int32)))
            best = jnp.min(jnp.where(m == jnp.max(m), mi, I32_BIG))
            plsc.store_scatter(res_v, [jnp.full((nl,), r, jnp.int32)],
                               jnp.full((nl,), best, jnp.int32),
                               mask=lane == 0)
            return carry

        lax.fori_loop(0, rows_per_w // 4, batch_body, 0)
        pltpu.sync_copy(res_v, out_hbm.at[pl.ds(base, rows_per_w)])

    return sc_kernel(y_train, w_train, grp_flat, dist)


def kernel(x_train, y_train, x_test, w_train):
    n = x_test.shape[0]
    h = n // 2
    outs = []
    for lo in (0, h):
        dist, grp = _topk_groups(x_train, lax.slice(x_test, (lo, 0),
                                                    (lo + h, x_test.shape[1])))
        outs.append(_vote_argmax(y_train, w_train, grp.reshape(-1), dist,
                                 h, 1024))
    return jnp.concatenate(outs)


# speedup vs baseline: 6.5098x; 1.0509x over previous
import functools

import jax
import jax.numpy as jnp
from jax import lax
from jax.experimental import pallas as pl
from jax.experimental.pallas import tpu as pltpu
from jax.experimental.pallas import tpu_sc as plsc

BR = 256
BC = 4096
G = 16
K = 16

F32_INF = float("inf")
I32_BIG = 1 << 30


def _group_body(x_test_ref, x_train_ref, dist_ref, grp_ref,
                gm_scratch, a2_scratch, b2_scratch):
    r = pl.program_id(0)
    c = pl.program_id(1)
    n_groups = gm_scratch.shape[1]
    gpb = BC // G

    a = x_test_ref[...]
    b = x_train_ref[...]

    @pl.when(c == 0)
    def _a_norms():
        a2_scratch[...] = jnp.sum(a * a, axis=1, keepdims=True)

    @pl.when(r == 0)
    def _b_norms():
        b2_scratch[:, pl.ds(c * BC, BC)] = jnp.sum(b * b, axis=1)[None, :]

    a2 = a2_scratch[...]
    b2 = b2_scratch[:, pl.ds(c * BC, BC)]
    mm = lax.dot_general(a, b, (((1,), (1,)), ((), ())),
                         preferred_element_type=jnp.float32)
    d2 = (a2 + b2) - 2.0 * mm
    dist = jnp.sqrt(jnp.maximum(d2, 0.0))
    dist_ref[...] = dist

    m = dist
    for s in (1, 2, 4, 8):
        m = jnp.minimum(m, pltpu.roll(m, BC - s, 1))
    io_l = lax.broadcasted_iota(jnp.int32, (BC, gpb), 0)
    io_g = lax.broadcasted_iota(jnp.int32, (BC, gpb), 1)
    sel = (io_l == G * io_g).astype(jnp.float32)
    gm = lax.dot_general(m, sel, (((1,), (0,)), ((), ())),
                         precision=lax.Precision.HIGHEST,
                         preferred_element_type=jnp.float32)
    gm_scratch[:, pl.ds(c * gpb, gpb)] = gm

    @pl.when(c == pl.num_programs(1) - 1)
    def _select():
        g = gm_scratch[...]
        iota_g = lax.broadcasted_iota(jnp.int32, (BR, n_groups), 1)
        thr = jnp.full((BR, 1), -F32_INF, jnp.float32)
        last = jnp.full((BR, 1), -1, jnp.int32)
        cols = []
        for _k in range(K):
            elig = (g > thr) | ((g == thr) & (iota_g > last))
            v = jnp.where(elig, g, F32_INF)
            cm = jnp.min(v, axis=1, keepdims=True)
            cmi = jnp.min(jnp.where(v == cm, iota_g, I32_BIG),
                          axis=1, keepdims=True)
            thr, last = cm, cmi
            cols.append(cmi)
        grp_ref[...] = jnp.concatenate(cols, axis=1)


def _topk_groups(x_train, x_test):
    n_test, d = x_test.shape
    n_train = x_train.shape[0]
    grid = (n_test // BR, n_train // BC)
    return pl.pallas_call(
        _group_body,
        grid=grid,
        in_specs=[
            pl.BlockSpec((BR, d), lambda r, c: (r, 0)),
            pl.BlockSpec((BC, d), lambda r, c: (c, 0)),
        ],
        out_specs=[
            pl.BlockSpec((BR, BC), lambda r, c: (r, c)),
            pl.BlockSpec((BR, K), lambda r, c: (r, 0)),
        ],
        out_shape=[
            jax.ShapeDtypeStruct((n_test, n_train), jnp.float32),
            jax.ShapeDtypeStruct((n_test, K), jnp.int32),
        ],
        scratch_shapes=[
            pltpu.VMEM((BR, n_train // G), jnp.float32),
            pltpu.VMEM((BR, 1), jnp.float32),
            pltpu.VMEM((1, n_train), jnp.float32),
        ],
        compiler_params=pltpu.CompilerParams(
            dimension_semantics=("arbitrary", "arbitrary")),
    )(x_test, x_train)


def _vote_argmax(y_train, w_train, grp_flat, dist, n_test, n_bins):
    info = plsc.get_sparse_core_info()
    nc, ns, nl = info.num_cores, info.num_subcores, info.num_lanes
    nw = nc * ns
    rows_per_w = n_test // nw
    n_train = y_train.shape[0]
    mesh = plsc.VectorSubcoreMesh(core_axis_name="c", subcore_axis_name="s")

    @functools.partial(
        pl.kernel, mesh=mesh,
        out_type=jax.ShapeDtypeStruct((n_test,), jnp.int32),
        compiler_params=pltpu.CompilerParams(needs_layout_passes=False),
        scratch_types=[
            pltpu.VMEM((n_train,), jnp.int32),
            pltpu.VMEM((n_train,), jnp.float32),
            pltpu.VMEM((rows_per_w * K,), jnp.int32),
            pltpu.VMEM((4, n_train), jnp.float32),
            pltpu.VMEM((n_bins,), jnp.float32),
            pltpu.VMEM((rows_per_w,), jnp.int32),
        ],
    )
    def sc_kernel(y_hbm, w_hbm, grp_hbm, dist_hbm, out_hbm,
                  y_v, w_v, grp_v, row_v, vote_v, res_v):
        wid = lax.axis_index("s") * nc + lax.axis_index("c")
        base = wid * rows_per_w
        pltpu.sync_copy(y_hbm, y_v)
        pltpu.sync_copy(w_hbm, w_v)
        pltpu.sync_copy(grp_hbm.at[pl.ds(base * K, rows_per_w * K)], grp_v)

        zeros16 = jnp.zeros((nl,), jnp.float32)
        lane = lax.broadcasted_iota(jnp.int32, (nl,), 0)

        def batch_body(bt, bcarry):
            pltpu.sync_copy(dist_hbm.at[pl.ds(base + bt * 4, 4)], row_v)
            lax.fori_loop(0, 4, lambda q, c2: row_body(bt * 4 + q, q, c2), 0)
            return bcarry

        def row_body(r, q, carry):
            def zero_body(i, zc):
                vote_v[pl.ds(i * nl, nl)] = zeros16
                return zc

            lax.fori_loop(0, n_bins // nl, zero_body, 0)

            tvecs, vals = [], []
            for j in range(K):
                gb = plsc.load_gather(
                    grp_v, [jnp.full((nl,), r * K + j, jnp.int32)])
                tvec = gb * G + lane
                tvecs.append(tvec)
                vals.append(plsc.load_gather(
                    row_v, [jnp.full((nl,), q, jnp.int32), tvec]))

            acc, _ = plsc.sort_key_val(vals[0], vals[0])
            cnt = jnp.zeros((nl,), jnp.int32)
            for j in range(1, K):
                cj, _ = plsc.sort_key_val(vals[j], vals[j])
                mrg = jnp.minimum(acc, lax.rev(cj, (0,)))
                acc, _ = plsc.sort_key_val(mrg, mrg)
            t = jnp.max(acc)

            for j in range(K):
                cnt = cnt + plsc.all_reduce_population_count(vals[j] <= t)
            n_le = jnp.max(cnt)

            @pl.when(n_le == K)
            def _common():
                for j in range(K):
                    lab = plsc.load_gather(y_v, [tvecs[j]])
                    wt = plsc.load_gather(w_v, [tvecs[j]])
                    plsc.addupdate_scatter(vote_v, [lab], wt,
                                           mask=vals[j] <= t)

            @pl.when(n_le != K)
            def _rare():
                def ext_body(_k, ecarry):
                    thr, lastg = ecarry
                    bv = jnp.full((nl,), F32_INF, jnp.float32)
                    bi = jnp.full((nl,), I32_BIG, jnp.int32)
                    for j in range(K):
                        v = vals[j]
                        tvec = tvecs[j]
                        elig = (v > thr) | ((v == thr) & (tvec > lastg))
                        vv = jnp.where(elig, v, F32_INF)
                        mv = jnp.min(vv)
                        mi = jnp.min(jnp.where(vv == mv, tvec, I32_BIG))
                        take = (mv < bv) | ((mv == bv) & (mi < bi))
                        bv = jnp.where(take, mv, bv)
                        bi = jnp.where(take, mi, bi)
                    lab = plsc.load_gather(y_v, [bi])
                    wt = plsc.load_gather(w_v, [bi])
                    plsc.addupdate_scatter(vote_v, [lab], wt,
                                           mask=lane == 0)
                    return (bv, bi)

                lax.fori_loop(0, K, ext_body,
                              (jnp.full((nl,), -F32_INF, jnp.float32),
                               jnp.full((nl,), -1, jnp.int32)))

            def amax_body(cb, mcarry):
                m, mi = mcarry
                v = vote_v[pl.ds(cb * nl, nl)]
                io = lane + cb * nl
                upd = v > m
                return (jnp.where(upd, v, m), jnp.where(upd, io, mi))

            m, mi = lax.fori_loop(
                0, n_bins // nl, amax_body,
                (jnp.full((nl,), -F32_INF, jnp.float32),
                 jnp.full((nl,), I32_BIG, jnp.int32)))
            best = jnp.min(jnp.where(m == jnp.max(m), mi, I32_BIG))
            plsc.store_scatter(res_v, [jnp.full((nl,), r, jnp.int32)],
                               jnp.full((nl,), best, jnp.int32),
                               mask=lane == 0)
            return carry

        lax.fori_loop(0, rows_per_w // 4, batch_body, 0)
        pltpu.sync_copy(res_v, out_hbm.at[pl.ds(base, rows_per_w)])

    return sc_kernel(y_train, w_train, grp_flat, dist)


def kernel(x_train, y_train, x_test, w_train):
    n = x_test.shape[0]
    h = n // 2
    outs = []
    for lo in (0, h):
        dist, grp = _topk_groups(x_train, lax.slice(x_test, (lo, 0),
                                                    (lo + h, x_test.shape[1])))
        outs.append(_vote_argmax(y_train, w_train, grp.reshape(-1), dist,
                                 h, 1024))
    return jnp.concatenate(outs)
